# baseline (device time: 128016 ns/iter reference)
import jax
import jax.numpy as jnp
from jax import lax
from jax.experimental import pallas as pl
from jax.experimental.pallas import tpu as pltpu

N_DEV = 4


def kernel(A, B):
    m, k = A.shape
    _, n = B.shape
    chunk = m // N_DEV
    half = chunk // 2

    def top(c):
        return pl.ds(c * chunk, half)

    def bot(c):
        return pl.ds(c * chunk + half, half)

    f32 = jnp.float32
    bf16 = jnp.bfloat16

    L, R, D = 0, 1, 2
    TOP, BOT = 0, 1

    def body(a_hbm, b_ref, out_ref,
             a_tile, rs_stage, rs_recv,
             a_sem, rs_ssem, rs_rsem, ag_ssem, ag_rsem, ack_sem):
        my_pos = lax.axis_index("i")

        def at(pos):
            return lax.rem(my_pos + N_DEV + pos, N_DEV)

        left, right, diag = at(-1), at(1), at(2)

        barrier_sem = pltpu.get_barrier_semaphore()
        for nbr in (left, right, diag):
            pl.semaphore_signal(
                barrier_sem, inc=1,
                device_id=(nbr,), device_id_type=pl.DeviceIdType.MESH,
            )
        pl.semaphore_wait(barrier_sem, 3)

        def fetch_a(c, slot):
            cp = pltpu.make_async_copy(
                a_hbm.at[pl.ds(c * chunk, chunk), :],
                a_tile.at[slot],
                a_sem.at[slot],
            )
            cp.start()
            return cp

        def compute(c, slot):
            out_ref[pl.ds(c * chunk, chunk), :] = jnp.dot(
                a_tile[slot], b_ref[:, :], preferred_element_type=f32,
            )

        sends = []

        def push_rs(rows, dest, peer_slot, half_slot, stage_slot):
            rs_stage[stage_slot] = out_ref[rows, :].astype(bf16)
            r = pltpu.make_async_remote_copy(
                src_ref=rs_stage.at[stage_slot],
                dst_ref=rs_recv.at[peer_slot, half_slot],
                send_sem=rs_ssem.at[stage_slot],
                recv_sem=rs_rsem.at[peer_slot, half_slot],
                device_id=(dest,), device_id_type=pl.DeviceIdType.MESH,
            )
            r.start()
            sends.append(r)
            return r

        f0 = fetch_a(at(0), 0)
        f1 = fetch_a(at(1), 1)
        f0.wait()
        compute(at(0), 0)
        s0 = push_rs(top(at(0)), left, R, TOP, 0)
        s1 = push_rs(bot(at(0)), right, L, BOT, 1)
        f2 = fetch_a(at(-1), 0)
        f1.wait()
        compute(at(1), 1)
        s2 = push_rs(bot(at(1)), diag, D, BOT, 2)
        f3 = fetch_a(at(2), 1)
        f2.wait()
        compute(at(-1), 0)
        s3 = push_rs(top(at(-1)), diag, D, TOP, 3)
        f3.wait()
        compute(at(2), 1)
        s0.wait_send()
        sends.remove(s0)
        push_rs(top(at(2)), right, L, TOP, 0)
        s1.wait_send()
        sends.remove(s1)
        push_rs(bot(at(2)), left, R, BOT, 1)

        def wait_rs(peer_slot, half_slot):
            pltpu.make_async_remote_copy(
                src_ref=rs_stage.at[0],
                dst_ref=rs_recv.at[peer_slot, half_slot],
                send_sem=rs_ssem.at[0],
                recv_sem=rs_rsem.at[peer_slot, half_slot],
                device_id=(left,), device_id_type=pl.DeviceIdType.MESH,
            ).wait_recv()

        for p in (L, R, D):
            wait_rs(p, TOP)
        out_ref[top(at(1)), :] = jnp.maximum(
            out_ref[top(at(1)), :]
            + rs_recv[L, TOP].astype(f32)
            + rs_recv[R, TOP].astype(f32)
            + rs_recv[D, TOP].astype(f32),
            0.0,
        )
        for p in (L, R, D):
            wait_rs(p, BOT)
        out_ref[bot(at(-1)), :] = jnp.maximum(
            out_ref[bot(at(-1)), :]
            + rs_recv[L, BOT].astype(f32)
            + rs_recv[R, BOT].astype(f32)
            + rs_recv[D, BOT].astype(f32),
            0.0,
        )

        for nbr in (left, right, diag):
            pl.semaphore_signal(
                ack_sem, inc=1,
                device_id=(nbr,), device_id_type=pl.DeviceIdType.MESH,
            )
        pl.semaphore_wait(ack_sem, 3)

        s2.wait_send()
        sends.remove(s2)
        s3.wait_send()
        sends.remove(s3)
        rs_stage[2] = out_ref[top(at(1)), :].astype(bf16)
        rs_stage[3] = out_ref[bot(at(-1)), :].astype(bf16)
        ag_pack = rs_stage.at[pl.ds(2, 2)]
        for idx, (dest, dst_peer) in enumerate(((left, R), (right, L), (diag, D))):
            r = pltpu.make_async_remote_copy(
                src_ref=ag_pack,
                dst_ref=rs_recv.at[dst_peer],
                send_sem=ag_ssem.at[idx],
                recv_sem=ag_rsem.at[dst_peer],
                device_id=(dest,), device_id_type=pl.DeviceIdType.MESH,
            )
            r.start()
            sends.append(r)

        def wait_ag(peer_slot):
            pltpu.make_async_remote_copy(
                src_ref=ag_pack, dst_ref=rs_recv.at[peer_slot],
                send_sem=ag_ssem.at[0], recv_sem=ag_rsem.at[peer_slot],
                device_id=(left,), device_id_type=pl.DeviceIdType.MESH,
            ).wait_recv()

        wait_ag(L)
        out_ref[top(at(0)), :] = rs_recv[L, TOP].astype(f32)
        out_ref[bot(at(2)), :] = rs_recv[L, BOT].astype(f32)
        wait_ag(R)
        out_ref[top(at(2)), :] = rs_recv[R, TOP].astype(f32)
        out_ref[bot(at(0)), :] = rs_recv[R, BOT].astype(f32)
        wait_ag(D)
        out_ref[top(at(-1)), :] = rs_recv[D, TOP].astype(f32)
        out_ref[bot(at(1)), :] = rs_recv[D, BOT].astype(f32)

        for r in sends:
            r.wait_send()

    return pl.pallas_call(
        body,
        out_shape=jax.ShapeDtypeStruct((m, n), f32),
        in_specs=[
            pl.BlockSpec(memory_space=pltpu.MemorySpace.HBM),
            pl.BlockSpec(memory_space=pltpu.VMEM),
        ],
        out_specs=pl.BlockSpec(memory_space=pltpu.VMEM),
        scratch_shapes=[
            pltpu.VMEM((2, chunk, k), f32),
            pltpu.VMEM((4, half, n), bf16),
            pltpu.VMEM((3, 2, half, n), bf16),
            pltpu.SemaphoreType.DMA((2,)),
            pltpu.SemaphoreType.DMA((4,)),
            pltpu.SemaphoreType.DMA((3, 2)),
            pltpu.SemaphoreType.DMA((3,)),
            pltpu.SemaphoreType.DMA((3,)),
            pltpu.SemaphoreType.REGULAR,
        ],
        compiler_params=pltpu.CompilerParams(
            collective_id=0,
            vmem_limit_bytes=40 * 1024 * 1024,
        ),
    )(A, B)
